# baseline (speedup 1.0000x reference)
import functools

import jax
import jax.numpy as jnp
import numpy as np
from jax import lax
from jax.experimental import pallas as pl
from jax.experimental.pallas import tpu as pltpu
from jax.experimental.pallas import tpu_sc as plsc

_H = 48
_W = 48
_T = _H * _W
_TT = 2304
_SUB = _T // 128
_OFFS = (-_W, -1, 1, _W)
_EPS = np.float32(1e-8)
_DENOM = np.float32(2.0 * 0.1 ** 2)
_NEG1 = np.float32(-1.0)


def _topk_body(cx_ref, cy_ref, cxs_ref, cys_ref, idx_ref):
    cx = cx_ref[...]
    cy = cy_ref[...]
    tok = (lax.broadcasted_iota(jnp.int32, (_SUB, 128), 0) * 128
           + lax.broadcasted_iota(jnp.int32, (_SUB, 128), 1))
    xcol = tok % _W
    val = [tok >= _W, xcol > 0, xcol < _W - 1, tok < _T - _W]
    sims = []
    cidx = []
    for j in range(4):
        dx = cx - cxs_ref[j]
        dy = cy - cys_ref[j]
        s = dx * dx + dy * dy
        dist = jnp.sqrt(s + _EPS)
        sim = jnp.exp(-(dist * dist) / _DENOM)
        sims.append(jnp.where(val[j], sim, _NEG1))
        cidx.append(tok + _OFFS[j])
    m1 = jnp.maximum(jnp.maximum(sims[0], sims[1]),
                     jnp.maximum(sims[2], sims[3]))
    idx1 = jnp.where(sims[0] == m1, cidx[0],
                     jnp.where(sims[1] == m1, cidx[1],
                               jnp.where(sims[2] == m1, cidx[2], cidx[3])))
    sims = [jnp.where(cidx[j] == idx1, _NEG1, sims[j]) for j in range(4)]
    m2 = jnp.maximum(jnp.maximum(sims[0], sims[1]),
                     jnp.maximum(sims[2], sims[3]))
    idx2 = jnp.where(sims[0] == m2, cidx[0],
                     jnp.where(sims[1] == m2, cidx[1],
                               jnp.where(sims[2] == m2, cidx[2], cidx[3])))
    idx_ref[0] = idx1
    idx_ref[1] = idx2


def _neighbor_idx(cx, cy):
    xg = cx.reshape(_H, _W)
    yg = cy.reshape(_H, _W)
    cxs = jnp.stack([jnp.roll(xg, 1, 0), jnp.roll(xg, 1, 1),
                     jnp.roll(xg, -1, 1), jnp.roll(xg, -1, 0)])
    cys = jnp.stack([jnp.roll(yg, 1, 0), jnp.roll(yg, 1, 1),
                     jnp.roll(yg, -1, 1), jnp.roll(yg, -1, 0)])
    out = pl.pallas_call(
        _topk_body,
        out_shape=jax.ShapeDtypeStruct((2, _SUB, 128), jnp.int32),
    )(cx.reshape(_SUB, 128), cy.reshape(_SUB, 128),
      cxs.reshape(4, _SUB, 128), cys.reshape(4, _SUB, 128))
    return out[0].reshape(_T), out[1].reshape(_T)


def _make_sc_gather(B, T, n_ch, chunk, n_chunks_per_worker, dtype):
    info = plsc.get_sparse_core_info()
    nc, ns = info.num_cores, info.num_subcores
    nw = nc * ns
    rpw = chunk * n_chunks_per_worker
    n_rows = nw * rpw
    mesh = plsc.VectorSubcoreMesh(core_axis_name="c", subcore_axis_name="s")

    @functools.partial(
        pl.kernel,
        mesh=mesh,
        out_type=jax.ShapeDtypeStruct((n_rows, n_ch), dtype),
        scratch_types=[
            pltpu.VMEM((n_chunks_per_worker, chunk), jnp.int32),
            pltpu.VMEM((rpw, n_ch), dtype),
            pltpu.SemaphoreType.DMA,
            pltpu.SemaphoreType.DMA,
        ],
    )
    def sc_gather(table_hbm, idx_hbm, out_hbm, idx_v, rows_v, sem, semw):
        wid = lax.axis_index("s") * nc + lax.axis_index("c")
        base = wid * rpw
        pltpu.sync_copy(idx_hbm.at[wid], idx_v)
        gathers = [
            pltpu.async_copy(table_hbm.at[idx_v.at[r]],
                             rows_v.at[pl.ds(r * chunk, chunk)], sem)
            for r in range(n_chunks_per_worker)
        ]
        for g in gathers:
            g.wait()
        pltpu.sync_copy(rows_v, out_hbm.at[pl.ds(base, rpw)])

    return sc_gather


def _conv_body(feat_ref, prime_ref, w0_ref, wk_ref, bias_ref, out_ref):
    dn = (((1,), (1,)), ((), ()))
    nb = feat_ref.shape[0]
    for bi in range(nb):
        acc = lax.dot_general(w0_ref[...], feat_ref[bi], dn,
                              preferred_element_type=jnp.float32)
        acc = acc + lax.dot_general(wk_ref[0], prime_ref[bi, 0], dn,
                                    preferred_element_type=jnp.float32)
        acc = acc + lax.dot_general(wk_ref[1], prime_ref[bi, 1], dn,
                                    preferred_element_type=jnp.float32)
        out_ref[bi] = acc + bias_ref[...]


def kernel(x, W, b):
    B, C, H, Wd = x.shape
    O = W.shape[0]
    T = H * Wd

    yv = jnp.linspace(-1.0, 1.0, H)
    xv = jnp.linspace(-1.0, 1.0, Wd)
    yg, xg = jnp.meshgrid(yv, xv, indexing="ij")
    cx = xg.reshape(T)
    cy = yg.reshape(T)

    idx1, idx2 = _neighbor_idx(cx, cy)

    feat = x.reshape(B, C, T)
    featT = jnp.swapaxes(feat, 1, 2)
    table = featT.reshape(B * T, C)

    chunk = 96
    n_workers = 32
    n_chunks_per_worker = 2 * B * T // (n_workers * chunk)
    off = (jnp.arange(B, dtype=jnp.int32) * T)[:, None, None]
    idx_kt = jnp.stack([idx1, idx2], axis=0)[None]
    gidx = (idx_kt + off).reshape(n_workers, n_chunks_per_worker, chunk)

    sc_gather = _make_sc_gather(B, T, C, chunk, n_chunks_per_worker,
                                jnp.float32)
    prime = sc_gather(table, gidx).reshape(B, 2, T, C)

    Wk = jnp.moveaxis(W, 2, 0)
    w0 = Wk[0]
    wk = Wk[1:]
    bias_col = b.reshape(O, 1)
    out = pl.pallas_call(
        _conv_body,
        out_shape=jax.ShapeDtypeStruct((B, O, T), jnp.float32),
    )(featT, prime, w0, wk, bias_col)
    return out.reshape(B, O, H, Wd)


# speedup vs baseline: 1.0231x; 1.0231x over previous
import functools

import jax
import jax.numpy as jnp
import numpy as np
from jax import lax
from jax.experimental import pallas as pl
from jax.experimental.pallas import tpu as pltpu
from jax.experimental.pallas import tpu_sc as plsc

_H = 48
_W = 48
_T = _H * _W
_TT = 2304
_SUB = _T // 128
_OFFS = (-_W, -1, 1, _W)
_EPS = np.float32(1e-8)
_DENOM = np.float32(2.0 * 0.1 ** 2)
_NEG1 = np.float32(-1.0)


def _topk_body(cx_ref, cy_ref, cxs_ref, cys_ref, idx_ref):
    cx = cx_ref[...]
    cy = cy_ref[...]
    tok = (lax.broadcasted_iota(jnp.int32, (_SUB, 128), 0) * 128
           + lax.broadcasted_iota(jnp.int32, (_SUB, 128), 1))
    xcol = tok % _W
    val = [tok >= _W, xcol > 0, xcol < _W - 1, tok < _T - _W]
    sims = []
    cidx = []
    for j in range(4):
        dx = cx - cxs_ref[j]
        dy = cy - cys_ref[j]
        s = dx * dx + dy * dy
        dist = jnp.sqrt(s + _EPS)
        sim = jnp.exp(-(dist * dist) / _DENOM)
        sims.append(jnp.where(val[j], sim, _NEG1))
        cidx.append(tok + _OFFS[j])
    m1 = jnp.maximum(jnp.maximum(sims[0], sims[1]),
                     jnp.maximum(sims[2], sims[3]))
    idx1 = jnp.where(sims[0] == m1, cidx[0],
                     jnp.where(sims[1] == m1, cidx[1],
                               jnp.where(sims[2] == m1, cidx[2], cidx[3])))
    sims = [jnp.where(cidx[j] == idx1, _NEG1, sims[j]) for j in range(4)]
    m2 = jnp.maximum(jnp.maximum(sims[0], sims[1]),
                     jnp.maximum(sims[2], sims[3]))
    idx2 = jnp.where(sims[0] == m2, cidx[0],
                     jnp.where(sims[1] == m2, cidx[1],
                               jnp.where(sims[2] == m2, cidx[2], cidx[3])))
    idx_ref[0] = idx1
    idx_ref[1] = idx2


def _neighbor_idx(cx, cy):
    xg = cx.reshape(_H, _W)
    yg = cy.reshape(_H, _W)
    cxs = jnp.stack([jnp.roll(xg, 1, 0), jnp.roll(xg, 1, 1),
                     jnp.roll(xg, -1, 1), jnp.roll(xg, -1, 0)])
    cys = jnp.stack([jnp.roll(yg, 1, 0), jnp.roll(yg, 1, 1),
                     jnp.roll(yg, -1, 1), jnp.roll(yg, -1, 0)])
    out = pl.pallas_call(
        _topk_body,
        out_shape=jax.ShapeDtypeStruct((2, _SUB, 128), jnp.int32),
    )(cx.reshape(_SUB, 128), cy.reshape(_SUB, 128),
      cxs.reshape(4, _SUB, 128), cys.reshape(4, _SUB, 128))
    return out[0].reshape(_T), out[1].reshape(_T)


def _make_sc_gather(B, T, n_ch, chunk, n_chunks_per_worker, dtype):
    info = plsc.get_sparse_core_info()
    nc, ns = info.num_cores, info.num_subcores
    nw = nc * ns
    rpw = chunk * n_chunks_per_worker
    n_rows = nw * rpw
    mesh = plsc.VectorSubcoreMesh(core_axis_name="c", subcore_axis_name="s")

    @functools.partial(
        pl.kernel,
        mesh=mesh,
        out_type=jax.ShapeDtypeStruct((n_rows, n_ch), dtype),
        scratch_types=[
            pltpu.VMEM((n_chunks_per_worker, chunk), jnp.int32),
            pltpu.VMEM((rpw, n_ch), dtype),
            pltpu.SemaphoreType.DMA,
        ],
    )
    def sc_gather(table_hbm, idx_hbm, out_hbm, idx_v, rows_v, sem):
        wid = lax.axis_index("s") * nc + lax.axis_index("c")
        base = wid * rpw
        pltpu.sync_copy(idx_hbm.at[wid], idx_v)
        gathers = [
            pltpu.async_copy(table_hbm.at[idx_v.at[r]],
                             rows_v.at[pl.ds(r * chunk, chunk)], sem)
            for r in range(n_chunks_per_worker)
        ]
        for g in gathers:
            g.wait()
        pltpu.sync_copy(rows_v, out_hbm.at[pl.ds(base, rpw)])

    return sc_gather


def _conv_body(feat_ref, prime_ref, w0_ref, wk_ref, bias_ref, out_ref):
    f0 = feat_ref[0]
    p1 = prime_ref[0, 0]
    p2 = prime_ref[0, 1]
    dn = (((1,), (1,)), ((), ()))
    acc = lax.dot_general(w0_ref[...], f0, dn,
                          preferred_element_type=jnp.float32)
    acc = acc + lax.dot_general(wk_ref[0], p1, dn,
                                preferred_element_type=jnp.float32)
    acc = acc + lax.dot_general(wk_ref[1], p2, dn,
                                preferred_element_type=jnp.float32)
    out_ref[0] = acc + bias_ref[...]


def kernel(x, W, b):
    B, C, H, Wd = x.shape
    O = W.shape[0]
    T = H * Wd

    yv = jnp.linspace(-1.0, 1.0, H)
    xv = jnp.linspace(-1.0, 1.0, Wd)
    yg, xg = jnp.meshgrid(yv, xv, indexing="ij")
    cx = xg.reshape(T)
    cy = yg.reshape(T)

    idx1, idx2 = _neighbor_idx(cx, cy)

    feat = x.reshape(B, C, T)
    featT = jnp.swapaxes(feat, 1, 2)
    table = featT.reshape(B * T, C)

    chunk = 96
    n_workers = 32
    n_chunks_per_worker = 2 * B * T // (n_workers * chunk)
    off = (jnp.arange(B, dtype=jnp.int32) * T)[:, None, None]
    idx_kt = jnp.stack([idx1, idx2], axis=0)[None]
    gidx = (idx_kt + off).reshape(n_workers, n_chunks_per_worker, chunk)

    sc_gather = _make_sc_gather(B, T, C, chunk, n_chunks_per_worker,
                                jnp.float32)
    prime = sc_gather(table, gidx).reshape(B, 2, T, C)

    Wk = jnp.moveaxis(W, 2, 0)
    w0 = Wk[0]
    wk = Wk[1:]
    bias_col = b.reshape(O, 1)
    out = pl.pallas_call(
        _conv_body,
        grid=(B, T // _TT),
        in_specs=[
            pl.BlockSpec((1, _TT, C), lambda bi, j: (bi, j, 0)),
            pl.BlockSpec((1, 2, _TT, C), lambda bi, j: (bi, 0, j, 0)),
            pl.BlockSpec((O, C), lambda bi, j: (0, 0)),
            pl.BlockSpec((2, O, C), lambda bi, j: (0, 0, 0)),
            pl.BlockSpec((O, 1), lambda bi, j: (0, 0)),
        ],
        out_specs=pl.BlockSpec((1, O, _TT), lambda bi, j: (bi, 0, j)),
        out_shape=jax.ShapeDtypeStruct((B, O, T), jnp.float32),
    )(featT, prime, w0, wk, bias_col)
    return out.reshape(B, O, H, Wd)
